# Initial kernel scaffold; baseline (speedup 1.0000x reference)
import jax
import jax.numpy as jnp
from jax.experimental import pallas as pl


def kernel(mol_x, motif_x, pro_x, pro_edge_weight, pro_emb, edge_attr, params, mol_edge_index, motif_edge_index, pro_edge_index, mol_batch, motif_batch, pro_batch):
    raise NotImplementedError("write your pallas kernel here")


# speedup vs baseline: 4.2924x; 4.2924x over previous
import functools

import jax
import jax.numpy as jnp
from jax import lax
from jax.experimental import pallas as pl
from jax.experimental.pallas import tpu as pltpu


def _ceil_to(x, m):
    return (x + m - 1) // m * m


def _pad2(x, m_to, n_to):
    M, N = x.shape
    if M == m_to and N == n_to:
        return x
    return jnp.pad(x, ((0, m_to - M), (0, n_to - N)))


def _mm_kernel(x_ref, w_ref, b_ref, o_ref, *, act, pre_relu):
    x = x_ref[...]
    if pre_relu:
        x = jnp.maximum(x, 0.0)
    acc = jnp.dot(x, w_ref[...], preferred_element_type=jnp.float32)
    acc = acc + b_ref[...]
    if act == "relu":
        acc = jnp.maximum(acc, 0.0)
    o_ref[...] = acc


def _mm(x, w, b=None, act=None, pre_relu=False, bm=512, bn=512):
    M, K = x.shape
    K2, N = w.shape
    assert K == K2, (x.shape, w.shape)
    if b is None:
        b = jnp.zeros((N,), jnp.float32)
    Kp = _ceil_to(K, 128)
    bm = min(bm, _ceil_to(M, 8))
    bn = min(bn, _ceil_to(N, 128))
    Mp = _ceil_to(M, bm)
    Np = _ceil_to(N, bn)
    xp = _pad2(x, Mp, Kp)
    wp = _pad2(w, Kp, Np)
    bp = jnp.pad(b, (0, Np - N)).reshape(1, Np)
    grid = (Mp // bm, Np // bn)
    out = pl.pallas_call(
        functools.partial(_mm_kernel, act=act, pre_relu=pre_relu),
        grid=grid,
        in_specs=[
            pl.BlockSpec((bm, Kp), lambda i, j: (i, 0)),
            pl.BlockSpec((Kp, bn), lambda i, j: (0, j)),
            pl.BlockSpec((1, bn), lambda i, j: (0, j)),
        ],
        out_specs=pl.BlockSpec((bm, bn), lambda i, j: (i, j)),
        out_shape=jax.ShapeDtypeStruct((Mp, Np), jnp.float32),
    )(xp, wp, bp)
    return out[:M, :N]


def _head_kernel(xm_ref, mm_ref, tp_ref, esm_ref, *refs):
    o_ref = refs[-1]
    (mf0w, mf0b, mf1w, mf1b, of0w, of0b, of1w, of1b,
     pf0w, pf0b, pf1w, pf1b, ef0w, ef0b, ef1w, ef1b,
     a0w, a0b, a1w, a1b, c0w, c0b, c1w, c1b, c2w, c2b) = [r[...] for r in refs[:-1]]

    def lin(z, w, b):
        return jnp.dot(z, w, preferred_element_type=jnp.float32) + b

    x = lin(jnp.maximum(xm_ref[...], 0.0), mf0w, mf0b)
    x = lin(jnp.maximum(x, 0.0), mf1w, mf1b)
    m = lin(jnp.maximum(mm_ref[...], 0.0), of0w, of0b)
    m = lin(jnp.maximum(m, 0.0), of1w, of1b)
    t = lin(jnp.maximum(tp_ref[...], 0.0), pf0w, pf0b)
    t = lin(jnp.maximum(t, 0.0), pf1w, pf1b)
    e = lin(jnp.maximum(esm_ref[...], 0.0), ef0w, ef0b)
    e = lin(e, ef1w, ef1b)
    fd = jnp.concatenate([x, m], axis=1)
    fp = jnp.concatenate([t, e], axis=1)

    def att(z):
        return lin(jnp.maximum(lin(z, a0w, a0b), 0.0), a1w, a1b)

    w1 = jax.nn.sigmoid(att(fd + fp))
    f1 = fd * w1 + fp * (1.0 - w1)
    w2 = jax.nn.sigmoid(att(f1))
    f2 = fd * w2 + fp * (1.0 - w2)
    c = jnp.maximum(lin(f2, c0w, c0b), 0.0)
    c = jnp.maximum(lin(c, c1w, c1b), 0.0)
    o_ref[...] = lin(c, c2w, c2b)


def _head(xm, mmo, tp, esm, P):
    ws = []
    for name in ("mol_fc", "motif_fc", "pro_fc", "esm_fc", "att", "cls"):
        for wgt, bia in P[name]:
            ws.append(wgt)
            ws.append(bia.reshape(1, -1))
    c2w, c2b = ws[-2], ws[-1]
    ws[-2] = jnp.pad(c2w, ((0, 0), (0, 127)))
    ws[-1] = jnp.pad(c2b, ((0, 0), (0, 127)))
    out = pl.pallas_call(
        _head_kernel,
        out_shape=jax.ShapeDtypeStruct((128, 128), jnp.float32),
    )(xm, mmo, tp, esm, *ws)
    return out[:, :1]


def _block_diag(a):
    H, O = a.shape
    eye = jnp.eye(H, dtype=a.dtype)
    return (a[:, :, None] * eye[:, None, :]).reshape(H * O, H)


def _gat_layer(x, src, dst, p, pre_relu):
    W, a_s, a_d, b = p
    inF, H, O = W.shape
    N = x.shape[0]
    if pre_relu:
        x = jnp.maximum(x, 0.0)
    Wflat = W.reshape(inF, H * O)
    Wes = _mm(Wflat, _block_diag(a_s))
    Wed = _mm(Wflat, _block_diag(a_d))
    es = _mm(x, Wes)
    ed = _mm(x, Wed)
    shift = jax.nn.leaky_relu(es.max(axis=0) + ed.max(axis=0), 0.2)
    e = jax.nn.leaky_relu(es[src] + ed[dst], 0.2)
    ex = jnp.exp(e - shift[None, :])
    ssum = jax.ops.segment_sum(ex, dst, num_segments=N)
    alpha = ex / ssum[dst]
    msg = (alpha[:, :, None] * x[src][:, None, :]).reshape(-1, H * inF)
    agg = jax.ops.segment_sum(msg, dst, num_segments=N)
    Wcat = jnp.transpose(W, (1, 0, 2)).reshape(H * inF, O) / H
    return _mm(agg, Wcat, b)


def _gcn_layer(x, src, dst, w_full, p):
    W, b = p
    N = x.shape[0]
    deg = jax.ops.segment_sum(w_full, dst, num_segments=N)
    dinv = jnp.where(deg > 0, deg ** -0.5, 0.0)
    norm = w_full * dinv[src] * dinv[dst]
    agg = jax.ops.segment_sum(norm[:, None] * x[src], dst, num_segments=N)
    return _mm(agg, W, b)


def _gmax(x, batch):
    out = jax.ops.segment_max(x, batch, num_segments=128)
    return jnp.where(jnp.isfinite(out), out, 0.0)


def _with_self_loops(ei, N):
    ar = jnp.arange(N, dtype=ei.dtype)
    return jnp.concatenate([ei[0], ar]), jnp.concatenate([ei[1], ar])


def kernel(mol_x, motif_x, pro_x, pro_edge_weight, pro_emb, edge_attr, params,
           mol_edge_index, motif_edge_index, pro_edge_index, mol_batch,
           motif_batch, pro_batch):
    P = params

    src, dst = _with_self_loops(mol_edge_index, mol_x.shape[0])
    x = _gat_layer(mol_x, src, dst, P["mol"][0], pre_relu=False)
    x = _gat_layer(x, src, dst, P["mol"][1], pre_relu=False)
    x = _gat_layer(x, src, dst, P["mol"][2], pre_relu=True)
    x = _gmax(jnp.maximum(x, 0.0), mol_batch)

    src, dst = _with_self_loops(motif_edge_index, motif_x.shape[0])
    m = _gat_layer(motif_x, src, dst, P["motif"][0], pre_relu=False)
    m = _gat_layer(m, src, dst, P["motif"][1], pre_relu=False)
    m = _gat_layer(m, src, dst, P["motif"][2], pre_relu=True)
    m = _gmax(jnp.maximum(m, 0.0), motif_batch)

    src, dst = _with_self_loops(pro_edge_index, pro_x.shape[0])
    w_full = jnp.concatenate(
        [pro_edge_weight, jnp.ones((pro_x.shape[0],), pro_x.dtype)])
    t = _gcn_layer(pro_x, src, dst, w_full, P["pro_gcn"])
    t = _gat_layer(t, src, dst, P["pro"][0], pre_relu=False)
    t = _gat_layer(t, src, dst, P["pro"][1], pre_relu=True)
    t = _gmax(jnp.maximum(t, 0.0), pro_batch)

    return _head(x, m, t, pro_emb, P)


# speedup vs baseline: 21.7178x; 5.0596x over previous
import functools

import jax
import jax.numpy as jnp
from jax import lax
from jax.experimental import pallas as pl
from jax.experimental.pallas import tpu as pltpu
from jax.experimental.pallas import tpu_sc as plsc

_NC, _NS, _NL = 2, 16, 16
_NW = _NC * _NS
_ROWS = 64


def _ceil_to(x, m):
    return (x + m - 1) // m * m


def _pad2(x, m_to, n_to):
    M, N = x.shape
    if M == m_to and N == n_to:
        return x
    return jnp.pad(x, ((0, m_to - M), (0, n_to - N)))


def _mm_kernel(x_ref, w_ref, b_ref, o_ref, *, act, pre_relu):
    x = x_ref[...]
    if pre_relu:
        x = jnp.maximum(x, 0.0)
    acc = jnp.dot(x, w_ref[...], preferred_element_type=jnp.float32)
    acc = acc + b_ref[...]
    if act == "relu":
        acc = jnp.maximum(acc, 0.0)
    o_ref[...] = acc


def _mm(x, w, b=None, act=None, pre_relu=False, bm=512, bn=512):
    M, K = x.shape
    K2, N = w.shape
    assert K == K2, (x.shape, w.shape)
    if b is None:
        b = jnp.zeros((N,), jnp.float32)
    Kp = _ceil_to(K, 128)
    bm = min(bm, _ceil_to(M, 8))
    bn = min(bn, _ceil_to(N, 128))
    Mp = _ceil_to(M, bm)
    Np = _ceil_to(N, bn)
    xp = _pad2(x, Mp, Kp)
    wp = _pad2(w, Kp, Np)
    bp = jnp.pad(b, (0, Np - N)).reshape(1, Np)
    grid = (Mp // bm, Np // bn)
    out = pl.pallas_call(
        functools.partial(_mm_kernel, act=act, pre_relu=pre_relu),
        grid=grid,
        in_specs=[
            pl.BlockSpec((bm, Kp), lambda i, j: (i, 0)),
            pl.BlockSpec((Kp, bn), lambda i, j: (0, j)),
            pl.BlockSpec((1, bn), lambda i, j: (0, j)),
        ],
        out_specs=pl.BlockSpec((bm, bn), lambda i, j: (i, j)),
        out_shape=jax.ShapeDtypeStruct((Mp, Np), jnp.float32),
    )(xp, wp, bp)
    return out[:M, :N]


_IOTA = None


def _iota16():
    return lax.iota(jnp.int32, 16)


def _bcast(s, dtype=jnp.int32):
    return jnp.full((16,), s, dtype)


def _lane(vec, j):
    return jnp.take(vec, jnp.full((16,), j % 16, jnp.int32), mode="wrap")


def _pick_ce(inFp, HinFp):
    budget = 480_000 - 64 * HinFp * 4 - 16_384
    per_edge = 76 + 64 + inFp * 4
    ce = (budget // per_edge) // 128 * 128
    return max(128, min(1024, ce))


@functools.lru_cache(maxsize=None)
def _gat_edge_kernel(N, H, inFp, NI):
    HinFp = H * inFp
    CE = _pick_ce(inFp, HinFp)
    NG = CE // 128
    IPW = -(-NI // _NW)
    NV = inFp // 16
    mesh = plsc.VectorSubcoreMesh(core_axis_name="c", subcore_axis_name="s")

    def body(es_hbm, ed_hbm, shift_hbm, x_hbm, src_hbm, dst_hbm, ibnd_hbm,
             agg_hbm, shift_v, bnd_v, sbuf, dbuf, idx_c, dst_c, es_rows,
             x_rows, ed_l, den, acc, sem):
        wid = lax.axis_index("s") * _NC + lax.axis_index("c")
        pltpu.sync_copy(shift_hbm, shift_v)
        es16 = shift_v[pl.ds(0, 16)]
        ed16 = shift_v[pl.ds(128, 16)]
        for k in range(1, 8):
            es16 = jnp.maximum(es16, shift_v[pl.ds(k * 16, 16)])
            ed16 = jnp.maximum(ed16, shift_v[pl.ds(128 + k * 16, 16)])
        sh = es16 + ed16
        sh = jnp.where(sh >= 0.0, sh, 0.2 * sh)
        shift = jnp.where(_iota16() < H, sh, 1e30)
        zeros = jnp.zeros((16,), jnp.float32)

        def leaky(v):
            return jnp.where(v >= 0.0, v, 0.2 * v)

        def stage_chunk(cb0, want_x):
            a0 = cb0 // 8 * 8
            off = cb0 - a0
            pltpu.sync_copy(src_hbm.at[pl.ds(a0, CE + 16)], sbuf)
            pltpu.sync_copy(dst_hbm.at[pl.ds(a0, CE + 16)], dbuf)
            for t in range(CE // 16):
                v = plsc.load_gather(sbuf, [_iota16() + _bcast(off + 16 * t)])
                idx_c[t // 8, pl.ds(t % 8 * 16, 16)] = v
                w = plsc.load_gather(dbuf, [_iota16() + _bcast(off + 16 * t)])
                dst_c[pl.ds(16 * t, 16)] = w
            cps = []
            for g in range(NG):
                cps.append(pltpu.async_copy(
                    es_hbm.at[idx_c.at[g]],
                    es_rows.at[pl.ds(g * 128, 128)], sem))
                if want_x:
                    cps.append(pltpu.async_copy(
                        x_hbm.at[idx_c.at[g]],
                        x_rows.at[pl.ds(g * 128, 128)], sem))
            for cp in cps:
                cp.wait()

        def item_body(it, car):
            item = wid * IPW + it

            @pl.when(item < NI)
            def _item():
                base = item * _ROWS
                pltpu.sync_copy(ibnd_hbm.at[pl.ds(item * 16, 16)], bnd_v)
                bv = bnd_v[pl.ds(0, 16)]
                b0 = bv[0]
                b1 = bv[1]
                ne = b1 - b0
                nchunks = (ne + CE - 1) // CE

                for q in range(_ROWS):
                    den[pl.ds(q * 16, 16)] = zeros

                def zbody(i, c):
                    acc[pl.ds(i * 16, 16)] = zeros
                    return c
                lax.fori_loop(0, _ROWS * HinFp // 16, zbody, 0)

                pltpu.sync_copy(ed_hbm.at[pl.ds(base * 16, _ROWS * 16)], ed_l)

                def edge_groups(cb0, nc, fn):
                    ngrp = (nc + 15) // 16

                    def gbody(g, c):
                        dvec = dst_c[pl.ds(g * 16, 16)]
                        for j in range(16):
                            eloc = g * 16 + j
                            dj = dvec[j]
                            dl = jnp.clip(dj - base, 0, _ROWS - 1)
                            validf = jnp.where(eloc < nc, 1.0, 0.0)
                            esr = plsc.load_gather(
                                es_rows, [_bcast(eloc), _iota16()])
                            edr = ed_l[pl.ds(dl * 16, 16)]
                            ex = jnp.exp(leaky(esr + edr) - shift) * validf
                            fn(eloc, dl, ex)
                        return c
                    lax.fori_loop(0, ngrp, gbody, 0)

                def p1_chunk(c, carry):
                    cb0 = b0 + c * CE
                    nc = jnp.minimum(b1 - cb0, CE)
                    stage_chunk(cb0, False)

                    def p1_edge(eloc, dl, ex):
                        plsc.addupdate(den.at[pl.ds(dl * 16, 16)], ex)
                    edge_groups(cb0, nc, p1_edge)
                    return carry
                lax.fori_loop(0, nchunks, p1_chunk, 0)

                for q in range(_ROWS):
                    d = den[pl.ds(q * 16, 16)]
                    den[pl.ds(q * 16, 16)] = 1.0 / d

                def p2_chunk(c, carry):
                    cb0 = b0 + c * CE
                    nc = jnp.minimum(b1 - cb0, CE)
                    stage_chunk(cb0, True)

                    def p2_edge(eloc, dl, ex):
                        rr = den[pl.ds(dl * 16, 16)]
                        al = ex * rr
                        xv = [plsc.load_gather(
                                  x_rows,
                                  [_bcast(eloc), _iota16() + _bcast(v * 16)])
                              for v in range(NV)]
                        ob = dl * HinFp
                        for h in range(H):
                            ah = _lane(al, h)
                            for v in range(NV):
                                plsc.addupdate(
                                    acc.at[pl.ds(ob + h * inFp + v * 16, 16)],
                                    ah * xv[v])
                    edge_groups(cb0, nc, p2_edge)
                    return carry
                lax.fori_loop(0, nchunks, p2_chunk, 0)

                pltpu.sync_copy(
                    acc, agg_hbm.at[pl.ds(base * HinFp, _ROWS * HinFp)])
            return car

        lax.fori_loop(0, IPW, item_body, 0)

    return pl.kernel(
        body,
        out_type=jax.ShapeDtypeStruct((NI * _ROWS * HinFp,), jnp.float32),
        mesh=mesh,
        compiler_params=pltpu.CompilerParams(
            needs_layout_passes=False, use_tc_tiling_on_sc=False),
        scratch_types=[
            pltpu.VMEM((256,), jnp.float32),
            pltpu.VMEM((16,), jnp.int32),
            pltpu.VMEM((CE + 16,), jnp.int32),
            pltpu.VMEM((CE + 16,), jnp.int32),
            pltpu.VMEM((NG, 128), jnp.int32),
            pltpu.VMEM((CE,), jnp.int32),
            pltpu.VMEM((CE, 16), jnp.float32),
            pltpu.VMEM((CE, inFp), jnp.float32),
            pltpu.VMEM((_ROWS * 16,), jnp.float32),
            pltpu.VMEM((_ROWS * 16,), jnp.float32),
            pltpu.VMEM((_ROWS * HinFp,), jnp.float32),
            pltpu.SemaphoreType.DMA,
        ],
    )


def _head_kernel(xm_ref, mm_ref, tp_ref, esm_ref, *refs):
    o_ref = refs[-1]
    (mf0w, mf0b, mf1w, mf1b, of0w, of0b, of1w, of1b,
     pf0w, pf0b, pf1w, pf1b, ef0w, ef0b, ef1w, ef1b,
     a0w, a0b, a1w, a1b, c0w, c0b, c1w, c1b, c2w, c2b) = [r[...] for r in refs[:-1]]

    def lin(z, w, b):
        return jnp.dot(z, w, preferred_element_type=jnp.float32) + b

    x = lin(jnp.maximum(xm_ref[...], 0.0), mf0w, mf0b)
    x = lin(jnp.maximum(x, 0.0), mf1w, mf1b)
    m = lin(jnp.maximum(mm_ref[...], 0.0), of0w, of0b)
    m = lin(jnp.maximum(m, 0.0), of1w, of1b)
    t = lin(jnp.maximum(tp_ref[...], 0.0), pf0w, pf0b)
    t = lin(jnp.maximum(t, 0.0), pf1w, pf1b)
    e = lin(jnp.maximum(esm_ref[...], 0.0), ef0w, ef0b)
    e = lin(e, ef1w, ef1b)
    fd = jnp.concatenate([x, m], axis=1)
    fp = jnp.concatenate([t, e], axis=1)

    def att(z):
        return lin(jnp.maximum(lin(z, a0w, a0b), 0.0), a1w, a1b)

    w1 = jax.nn.sigmoid(att(fd + fp))
    f1 = fd * w1 + fp * (1.0 - w1)
    w2 = jax.nn.sigmoid(att(f1))
    f2 = fd * w2 + fp * (1.0 - w2)
    c = jnp.maximum(lin(f2, c0w, c0b), 0.0)
    c = jnp.maximum(lin(c, c1w, c1b), 0.0)
    o_ref[...] = lin(c, c2w, c2b)


def _head(xm, mmo, tp, esm, P):
    ws = []
    for name in ("mol_fc", "motif_fc", "pro_fc", "esm_fc", "att", "cls"):
        for wgt, bia in P[name]:
            ws.append(wgt)
            ws.append(bia.reshape(1, -1))
    c2w, c2b = ws[-2], ws[-1]
    ws[-2] = jnp.pad(c2w, ((0, 0), (0, 127)))
    ws[-1] = jnp.pad(c2b, ((0, 0), (0, 127)))
    out = pl.pallas_call(
        _head_kernel,
        out_shape=jax.ShapeDtypeStruct((128, 128), jnp.float32),
    )(xm, mmo, tp, esm, *ws)
    return out[:, :1]


def _block_diag(a):
    H, O = a.shape
    eye = jnp.eye(H, dtype=a.dtype)
    return (a[:, :, None] * eye[:, None, :]).reshape(H * O, H)


def _shift_kernel(a_ref, b_ref, o_ref):
    ma = jnp.max(a_ref[...], axis=0, keepdims=True)
    mb = jnp.max(b_ref[...], axis=0, keepdims=True)
    o_ref[...] = jnp.concatenate([ma, mb], axis=0)


def _shift(es_p, ed_p):
    esr = es_p.reshape(-1, 128)
    edr = ed_p.reshape(-1, 128)
    out = pl.pallas_call(
        _shift_kernel,
        out_shape=jax.ShapeDtypeStruct((2, 128), jnp.float32),
    )(esr, edr)
    return out.reshape(256)


def _edge_prep(ei, N, ew=None):
    NI = -(-N // _ROWS)
    ar = jnp.arange(N, dtype=jnp.int32)
    src = jnp.concatenate([ei[0].astype(jnp.int32), ar])
    dst = jnp.concatenate([ei[1].astype(jnp.int32), ar])
    ops = [dst, src]
    if ew is not None:
        ops.append(jnp.concatenate([ew, jnp.ones((N,), jnp.float32)]))
    sorted_ops = lax.sort(ops, num_keys=1)
    dst_s, src_s = sorted_ops[0], sorted_ops[1]
    bounds = jnp.arange(NI + 1, dtype=jnp.int32) * _ROWS
    nbnd = jnp.searchsorted(dst_s, bounds).astype(jnp.int32)
    ibnd = jnp.zeros((NI, 16), jnp.int32)
    ibnd = ibnd.at[:, 0].set(nbnd[:-1]).at[:, 1].set(nbnd[1:]).reshape(-1)
    src_p = jnp.pad(src_s, (0, 1216))
    dst_p = jnp.pad(dst_s, (0, 1216))
    ew_s = None
    if ew is not None:
        ew_s = jnp.pad(sorted_ops[2], (0, 1216))
    return src_p, dst_p, ibnd, NI, ew_s


def _gat_layer(x, prep, p, out_relu):
    W, a_s, a_d, b = p
    inF, H, O = W.shape
    N = x.shape[0]
    src_s, dst_s, nbnd, NI, _ = prep
    inFp = _ceil_to(inF, 16)
    Npad = NI * _ROWS
    Wflat = W.reshape(inF, H * O)
    Wes = _mm(Wflat, _block_diag(a_s))
    Wed = _mm(Wflat, _block_diag(a_d))
    es = _mm(x, jnp.pad(Wes, ((0, 0), (0, 16 - H))))
    ed = _mm(x, jnp.pad(Wed, ((0, 0), (0, 16 - H))))
    es_p = jnp.pad(es, ((0, Npad - N), (0, 0)))
    ed_p = jnp.pad(ed, ((0, Npad - N), (0, 0))).reshape(-1)
    shift = _shift(es_p, ed_p)
    x_p = jnp.pad(x, ((0, 0), (0, inFp - inF)))
    agg = _gat_edge_kernel(N, H, inFp, NI)(
        es_p, ed_p, shift, x_p, src_s, dst_s, nbnd)
    agg = agg.reshape(Npad, H * inFp)[:N]
    Wp = jnp.pad(W, ((0, inFp - inF), (0, 0), (0, 0)))
    Wcat = jnp.transpose(Wp, (1, 0, 2)).reshape(H * inFp, O) / H
    return _mm(agg, Wcat, b, act="relu" if out_relu else None)


def _gcn_layer(x, src, dst, w_full, p):
    W, b = p
    N = x.shape[0]
    deg = jax.ops.segment_sum(w_full, dst, num_segments=N)
    dinv = jnp.where(deg > 0, deg ** -0.5, 0.0)
    norm = w_full * dinv[src] * dinv[dst]
    agg = jax.ops.segment_sum(norm[:, None] * x[src], dst, num_segments=N)
    return _mm(agg, W, b)


def _gmax(x, batch):
    out = jax.ops.segment_max(x, batch, num_segments=128)
    return jnp.where(jnp.isfinite(out), out, 0.0)


def kernel(mol_x, motif_x, pro_x, pro_edge_weight, pro_emb, edge_attr, params,
           mol_edge_index, motif_edge_index, pro_edge_index, mol_batch,
           motif_batch, pro_batch):
    P = params

    prep = _edge_prep(mol_edge_index, mol_x.shape[0])
    x = _gat_layer(mol_x, prep, P["mol"][0], out_relu=False)
    x = _gat_layer(x, prep, P["mol"][1], out_relu=True)
    x = _gat_layer(x, prep, P["mol"][2], out_relu=True)
    x = _gmax(x, mol_batch)

    prep = _edge_prep(motif_edge_index, motif_x.shape[0])
    m = _gat_layer(motif_x, prep, P["motif"][0], out_relu=False)
    m = _gat_layer(m, prep, P["motif"][1], out_relu=True)
    m = _gat_layer(m, prep, P["motif"][2], out_relu=True)
    m = _gmax(m, motif_batch)

    N = pro_x.shape[0]
    ar = jnp.arange(N, dtype=jnp.int32)
    src = jnp.concatenate([pro_edge_index[0].astype(jnp.int32), ar])
    dst = jnp.concatenate([pro_edge_index[1].astype(jnp.int32), ar])
    w_full = jnp.concatenate([pro_edge_weight, jnp.ones((N,), pro_x.dtype)])
    t = _gcn_layer(pro_x, src, dst, w_full, P["pro_gcn"])
    prep = _edge_prep(pro_edge_index, N)
    t = _gat_layer(t, prep, P["pro"][0], out_relu=True)
    t = _gat_layer(t, prep, P["pro"][1], out_relu=True)
    t = _gmax(t, pro_batch)

    return _head(x, m, t, pro_emb, P)


# speedup vs baseline: 33.1759x; 1.5276x over previous
import functools

import jax
import jax.numpy as jnp
from jax import lax
from jax.experimental import pallas as pl
from jax.experimental.pallas import tpu as pltpu
from jax.experimental.pallas import tpu_sc as plsc

_NC, _NS, _NL = 2, 16, 16
_NW = _NC * _NS
_ROWS = 64


def _ceil_to(x, m):
    return (x + m - 1) // m * m


def _pad2(x, m_to, n_to):
    M, N = x.shape
    if M == m_to and N == n_to:
        return x
    return jnp.pad(x, ((0, m_to - M), (0, n_to - N)))


def _mm_kernel(x_ref, w_ref, b_ref, o_ref, *, act, pre_relu):
    x = x_ref[...]
    if pre_relu:
        x = jnp.maximum(x, 0.0)
    acc = jnp.dot(x, w_ref[...], preferred_element_type=jnp.float32)
    acc = acc + b_ref[...]
    if act == "relu":
        acc = jnp.maximum(acc, 0.0)
    o_ref[...] = acc


def _mm(x, w, b=None, act=None, pre_relu=False, bm=512, bn=512):
    M, K = x.shape
    K2, N = w.shape
    assert K == K2, (x.shape, w.shape)
    if b is None:
        b = jnp.zeros((N,), jnp.float32)
    Kp = _ceil_to(K, 128)
    bm = min(bm, _ceil_to(M, 8))
    bn = min(bn, _ceil_to(N, 128))
    Mp = _ceil_to(M, bm)
    Np = _ceil_to(N, bn)
    xp = _pad2(x, Mp, Kp)
    wp = _pad2(w, Kp, Np)
    bp = jnp.pad(b, (0, Np - N)).reshape(1, Np)
    grid = (Mp // bm, Np // bn)
    out = pl.pallas_call(
        functools.partial(_mm_kernel, act=act, pre_relu=pre_relu),
        grid=grid,
        in_specs=[
            pl.BlockSpec((bm, Kp), lambda i, j: (i, 0)),
            pl.BlockSpec((Kp, bn), lambda i, j: (0, j)),
            pl.BlockSpec((1, bn), lambda i, j: (0, j)),
        ],
        out_specs=pl.BlockSpec((bm, bn), lambda i, j: (i, j)),
        out_shape=jax.ShapeDtypeStruct((Mp, Np), jnp.float32),
    )(xp, wp, bp)
    return out[:M, :N]


_IOTA = None


def _iota16():
    return lax.iota(jnp.int32, 16)


def _bcast(s, dtype=jnp.int32):
    return jnp.full((16,), s, dtype)


def _lane(vec, j):
    return jnp.take(vec, jnp.full((16,), j % 16, jnp.int32), mode="wrap")


def _pick_ce(inFp, HinFp):
    budget = 480_000 - 64 * HinFp * 4 - 16_384
    per_edge = 76 + 64 + inFp * 4
    ce = (budget // per_edge) // 128 * 128
    return max(128, min(1024, ce))


@functools.lru_cache(maxsize=None)
def _gat_edge_kernel(N, H, inFp, NI):
    HinFp = H * inFp
    CE = _pick_ce(inFp, HinFp)
    NG = CE // 128
    IPW = -(-NI // _NW)
    NV = inFp // 16
    mesh = plsc.VectorSubcoreMesh(core_axis_name="c", subcore_axis_name="s")

    def body(es_hbm, ed_hbm, shift_hbm, x_hbm, src_hbm, dst_hbm, ibnd_hbm,
             agg_hbm, shift_v, bnd_v, sbuf, dbuf, idx_c, dst_c, es_rows,
             x_rows, ed_l, den, acc, sem):
        wid = lax.axis_index("s") * _NC + lax.axis_index("c")
        pltpu.sync_copy(shift_hbm, shift_v)
        es16 = shift_v[pl.ds(0, 16)]
        ed16 = shift_v[pl.ds(128, 16)]
        for k in range(1, 8):
            es16 = jnp.maximum(es16, shift_v[pl.ds(k * 16, 16)])
            ed16 = jnp.maximum(ed16, shift_v[pl.ds(128 + k * 16, 16)])
        sh = es16 + ed16
        sh = jnp.where(sh >= 0.0, sh, 0.2 * sh)
        shift = jnp.where(_iota16() < H, sh, 1e30)
        zeros = jnp.zeros((16,), jnp.float32)

        def leaky(v):
            return jnp.where(v >= 0.0, v, 0.2 * v)

        def stage_chunk(cb0, want_x):
            a0 = cb0 // 8 * 8
            off = cb0 - a0
            pltpu.sync_copy(src_hbm.at[pl.ds(a0, CE + 16)], sbuf)
            pltpu.sync_copy(dst_hbm.at[pl.ds(a0, CE + 16)], dbuf)
            for t in range(CE // 16):
                v = plsc.load_gather(sbuf, [_iota16() + _bcast(off + 16 * t)])
                idx_c[t // 8, pl.ds(t % 8 * 16, 16)] = v
                w = plsc.load_gather(dbuf, [_iota16() + _bcast(off + 16 * t)])
                dst_c[pl.ds(16 * t, 16)] = w
            cps = []
            for g in range(NG):
                cps.append(pltpu.async_copy(
                    es_hbm.at[idx_c.at[g]],
                    es_rows.at[pl.ds(g * 128, 128)], sem))
                if want_x:
                    cps.append(pltpu.async_copy(
                        x_hbm.at[idx_c.at[g]],
                        x_rows.at[pl.ds(g * 128, 128)], sem))
            for cp in cps:
                cp.wait()

        def item_body(it, car):
            item = wid * IPW + it

            @pl.when(item < NI)
            def _item():
                base = item * _ROWS
                pltpu.sync_copy(ibnd_hbm.at[pl.ds(item * 16, 16)], bnd_v)
                bv = bnd_v[pl.ds(0, 16)]
                b0 = bv[0]
                b1 = bv[1]
                ne = b1 - b0
                nchunks = (ne + CE - 1) // CE

                for q in range(_ROWS):
                    den[pl.ds(q * 16, 16)] = zeros

                def zbody(i, c):
                    acc[pl.ds(i * 16, 16)] = zeros
                    return c
                lax.fori_loop(0, _ROWS * HinFp // 16, zbody, 0)

                pltpu.sync_copy(ed_hbm.at[pl.ds(base * 16, _ROWS * 16)], ed_l)

                def edge_groups(cb0, nc, fn):
                    ngrp = (nc + 15) // 16

                    def gbody(g, c):
                        dvec = dst_c[pl.ds(g * 16, 16)]
                        for j in range(16):
                            eloc = g * 16 + j
                            dj = dvec[j]
                            dl = jnp.clip(dj - base, 0, _ROWS - 1)
                            validf = jnp.where(eloc < nc, 1.0, 0.0)
                            esr = plsc.load_gather(
                                es_rows, [_bcast(eloc), _iota16()])
                            edr = ed_l[pl.ds(dl * 16, 16)]
                            ex = jnp.exp(leaky(esr + edr) - shift) * validf
                            fn(eloc, dl, ex)
                        return c
                    lax.fori_loop(0, ngrp, gbody, 0)

                def p1_chunk(c, carry):
                    cb0 = b0 + c * CE
                    nc = jnp.minimum(b1 - cb0, CE)
                    stage_chunk(cb0, False)

                    def p1_edge(eloc, dl, ex):
                        plsc.addupdate(den.at[pl.ds(dl * 16, 16)], ex)
                    edge_groups(cb0, nc, p1_edge)
                    return carry
                lax.fori_loop(0, nchunks, p1_chunk, 0)

                for q in range(_ROWS):
                    d = den[pl.ds(q * 16, 16)]
                    den[pl.ds(q * 16, 16)] = 1.0 / d

                def p2_chunk(c, carry):
                    cb0 = b0 + c * CE
                    nc = jnp.minimum(b1 - cb0, CE)
                    stage_chunk(cb0, True)

                    def p2_edge(eloc, dl, ex):
                        rr = den[pl.ds(dl * 16, 16)]
                        al = ex * rr
                        xv = [plsc.load_gather(
                                  x_rows,
                                  [_bcast(eloc), _iota16() + _bcast(v * 16)])
                              for v in range(NV)]
                        ob = dl * HinFp
                        for h in range(H):
                            ah = _lane(al, h)
                            for v in range(NV):
                                plsc.addupdate(
                                    acc.at[pl.ds(ob + h * inFp + v * 16, 16)],
                                    ah * xv[v])
                    edge_groups(cb0, nc, p2_edge)
                    return carry
                lax.fori_loop(0, nchunks, p2_chunk, 0)

                pltpu.sync_copy(
                    acc, agg_hbm.at[pl.ds(base * HinFp, _ROWS * HinFp)])
            return car

        lax.fori_loop(0, IPW, item_body, 0)

    return pl.kernel(
        body,
        out_type=jax.ShapeDtypeStruct((NI * _ROWS * HinFp,), jnp.float32),
        mesh=mesh,
        compiler_params=pltpu.CompilerParams(
            needs_layout_passes=False, use_tc_tiling_on_sc=False),
        scratch_types=[
            pltpu.VMEM((256,), jnp.float32),
            pltpu.VMEM((16,), jnp.int32),
            pltpu.VMEM((CE + 16,), jnp.int32),
            pltpu.VMEM((CE + 16,), jnp.int32),
            pltpu.VMEM((NG, 128), jnp.int32),
            pltpu.VMEM((CE,), jnp.int32),
            pltpu.VMEM((CE, 16), jnp.float32),
            pltpu.VMEM((CE, inFp), jnp.float32),
            pltpu.VMEM((_ROWS * 16,), jnp.float32),
            pltpu.VMEM((_ROWS * 16,), jnp.float32),
            pltpu.VMEM((_ROWS * HinFp,), jnp.float32),
            pltpu.SemaphoreType.DMA,
        ],
    )


def _rsqrt16(x):
    i = plsc.bitcast(x, jnp.int32)
    i = jnp.full((16,), 0x5F3759DF, jnp.int32) - (i >> 1)
    y = plsc.bitcast(i, jnp.float32)
    for _ in range(3):
        y = y * (1.5 - 0.5 * x * y * y)
    return y


@functools.lru_cache(maxsize=None)
def _gcn_deg_kernel(N, NI):
    CE = 1024
    IPW = -(-NI // _NW)
    mesh = plsc.VectorSubcoreMesh(core_axis_name="c", subcore_axis_name="s")

    def body(w_hbm, dst_hbm, ibnd_hbm, dinv_hbm, bnd_v, wbuf, dbuf, w_c,
             dst_c, den):
        wid = lax.axis_index("s") * _NC + lax.axis_index("c")
        zeros = jnp.zeros((16,), jnp.float32)

        def item_body(it, car):
            item = wid * IPW + it

            @pl.when(item < NI)
            def _item():
                base = item * _ROWS
                pltpu.sync_copy(ibnd_hbm.at[pl.ds(item * 16, 16)], bnd_v)
                bv = bnd_v[pl.ds(0, 16)]
                b0 = bv[0]
                b1 = bv[1]
                nchunks = (b1 - b0 + CE - 1) // CE
                for q in range(_ROWS):
                    den[pl.ds(q * 16, 16)] = zeros

                def chunk(c, carry):
                    cb0 = b0 + c * CE
                    nc = jnp.minimum(b1 - cb0, CE)
                    a0 = cb0 // 8 * 8
                    off = cb0 - a0
                    pltpu.sync_copy(w_hbm.at[pl.ds(a0, CE + 16)], wbuf)
                    pltpu.sync_copy(dst_hbm.at[pl.ds(a0, CE + 16)], dbuf)
                    for t in range(CE // 16):
                        ix = _iota16() + _bcast(off + 16 * t)
                        w_c[pl.ds(16 * t, 16)] = plsc.load_gather(wbuf, [ix])
                        dst_c[pl.ds(16 * t, 16)] = plsc.load_gather(dbuf, [ix])

                    def gbody(g, cc):
                        wvec = w_c[pl.ds(g * 16, 16)]
                        dvec = dst_c[pl.ds(g * 16, 16)]
                        for j in range(16):
                            eloc = g * 16 + j
                            dl = jnp.clip(dvec[j] - base, 0, _ROWS - 1)
                            validf = jnp.where(eloc < nc, 1.0, 0.0)
                            plsc.addupdate(
                                den.at[pl.ds(dl * 16, 16)],
                                jnp.full((16,), wvec[j], jnp.float32) * validf)
                        return cc
                    lax.fori_loop(0, (nc + 15) // 16, gbody, 0)
                    return carry
                lax.fori_loop(0, nchunks, chunk, 0)

                for q in range(_ROWS):
                    d = den[pl.ds(q * 16, 16)]
                    den[pl.ds(q * 16, 16)] = jnp.where(
                        d > 0.0, _rsqrt16(d), zeros)
                pltpu.sync_copy(den, dinv_hbm.at[pl.ds(base * 16, _ROWS * 16)])
            return car

        lax.fori_loop(0, IPW, item_body, 0)

    return pl.kernel(
        body,
        out_type=jax.ShapeDtypeStruct((NI * _ROWS * 16,), jnp.float32),
        mesh=mesh,
        compiler_params=pltpu.CompilerParams(
            needs_layout_passes=False, use_tc_tiling_on_sc=False),
        scratch_types=[
            pltpu.VMEM((16,), jnp.int32),
            pltpu.VMEM((CE + 16,), jnp.float32),
            pltpu.VMEM((CE + 16,), jnp.int32),
            pltpu.VMEM((CE,), jnp.float32),
            pltpu.VMEM((CE,), jnp.int32),
            pltpu.VMEM((_ROWS * 16,), jnp.float32),
        ],
    )


@functools.lru_cache(maxsize=None)
def _gcn_agg_kernel(N, inFp, NI):
    CE = 512
    NG = CE // 128
    IPW = -(-NI // _NW)
    NV = inFp // 16
    mesh = plsc.VectorSubcoreMesh(core_axis_name="c", subcore_axis_name="s")

    def body(dinv2_hbm, x_hbm, w_hbm, src_hbm, dst_hbm, ibnd_hbm,
             agg_hbm, bnd_v, sbuf, wbuf, dbuf, idx_c, w_c, dst_c, di_rows,
             x_rows, di_l, acc, sem):
        wid = lax.axis_index("s") * _NC + lax.axis_index("c")
        zeros = jnp.zeros((16,), jnp.float32)

        def item_body(it, car):
            item = wid * IPW + it

            @pl.when(item < NI)
            def _item():
                base = item * _ROWS
                pltpu.sync_copy(ibnd_hbm.at[pl.ds(item * 16, 16)], bnd_v)
                bv = bnd_v[pl.ds(0, 16)]
                b0 = bv[0]
                b1 = bv[1]
                nchunks = (b1 - b0 + CE - 1) // CE

                def zbody(i, c):
                    acc[pl.ds(i * 16, 16)] = zeros
                    return c
                lax.fori_loop(0, _ROWS * inFp // 16, zbody, 0)
                pltpu.sync_copy(dinv2_hbm.at[pl.ds(base, _ROWS)], di_l)

                def chunk(c, carry):
                    cb0 = b0 + c * CE
                    nc = jnp.minimum(b1 - cb0, CE)
                    a0 = cb0 // 8 * 8
                    off = cb0 - a0
                    pltpu.sync_copy(src_hbm.at[pl.ds(a0, CE + 16)], sbuf)
                    pltpu.sync_copy(w_hbm.at[pl.ds(a0, CE + 16)], wbuf)
                    pltpu.sync_copy(dst_hbm.at[pl.ds(a0, CE + 16)], dbuf)
                    for t in range(CE // 16):
                        ix = _iota16() + _bcast(off + 16 * t)
                        idx_c[t // 8, pl.ds(t % 8 * 16, 16)] = (
                            plsc.load_gather(sbuf, [ix]))
                        w_c[pl.ds(16 * t, 16)] = plsc.load_gather(wbuf, [ix])
                        dst_c[pl.ds(16 * t, 16)] = plsc.load_gather(dbuf, [ix])
                    cps = []
                    for g in range(NG):
                        cps.append(pltpu.async_copy(
                            dinv2_hbm.at[idx_c.at[g]],
                            di_rows.at[pl.ds(g * 128, 128)], sem))
                        cps.append(pltpu.async_copy(
                            x_hbm.at[idx_c.at[g]],
                            x_rows.at[pl.ds(g * 128, 128)], sem))
                    for cp in cps:
                        cp.wait()

                    def gbody(g, cc):
                        wvec = w_c[pl.ds(g * 16, 16)]
                        dvec = dst_c[pl.ds(g * 16, 16)]
                        for j in range(16):
                            eloc = g * 16 + j
                            dl = jnp.clip(dvec[j] - base, 0, _ROWS - 1)
                            validf = jnp.where(eloc < nc, 1.0, 0.0)
                            ds_row = plsc.load_gather(
                                di_rows, [_bcast(eloc), _iota16()])
                            dd_row = plsc.load_gather(
                                di_l, [_bcast(dl), _iota16()])
                            al = (jnp.full((16,), wvec[j], jnp.float32)
                                  * ds_row * dd_row * validf)
                            for v in range(NV):
                                xv = plsc.load_gather(
                                    x_rows,
                                    [_bcast(eloc), _iota16() + _bcast(v * 16)])
                                plsc.addupdate(
                                    acc.at[pl.ds(dl * inFp + v * 16, 16)],
                                    al * xv)
                        return cc
                    lax.fori_loop(0, (nc + 15) // 16, gbody, 0)
                    return carry
                lax.fori_loop(0, nchunks, chunk, 0)
                pltpu.sync_copy(
                    acc, agg_hbm.at[pl.ds(base * inFp, _ROWS * inFp)])
            return car

        lax.fori_loop(0, IPW, item_body, 0)

    return pl.kernel(
        body,
        out_type=jax.ShapeDtypeStruct((NI * _ROWS * inFp,), jnp.float32),
        mesh=mesh,
        compiler_params=pltpu.CompilerParams(
            needs_layout_passes=False, use_tc_tiling_on_sc=False),
        scratch_types=[
            pltpu.VMEM((16,), jnp.int32),
            pltpu.VMEM((CE + 16,), jnp.int32),
            pltpu.VMEM((CE + 16,), jnp.float32),
            pltpu.VMEM((CE + 16,), jnp.int32),
            pltpu.VMEM((NG, 128), jnp.int32),
            pltpu.VMEM((CE,), jnp.float32),
            pltpu.VMEM((CE,), jnp.int32),
            pltpu.VMEM((CE, 16), jnp.float32),
            pltpu.VMEM((CE, inFp), jnp.float32),
            pltpu.VMEM((_ROWS, 16), jnp.float32),
            pltpu.VMEM((_ROWS * inFp,), jnp.float32),
            pltpu.SemaphoreType.DMA,
        ],
    )


@functools.lru_cache(maxsize=None)
def _gmax_kernel(N, W):
    NV = W // 16
    mesh = plsc.VectorSubcoreMesh(core_axis_name="c", subcore_axis_name="s")

    def body(x_hbm, gbnd_hbm, out_hbm, bnd_v, xb, ob, sem):
        wid = lax.axis_index("s") * _NC + lax.axis_index("c")
        zeros = jnp.zeros((16,), jnp.float32)

        for i in range(128 // _NW):
            g = wid * (128 // _NW) + i
            pltpu.sync_copy(gbnd_hbm.at[pl.ds(g * 16, 16)], bnd_v)
            bv = bnd_v[pl.ds(0, 16)]
            r0 = bv[0]
            r1 = bv[1]
            r0a = r0 // 8 * 8
            nb = (r1 - r0a + 15) // 16

            def bbody(bidx, carry):
                rb = r0a + bidx * 16
                pltpu.sync_copy(x_hbm.at[pl.ds(rb, 16)], xb)
                out = list(carry)
                for j in range(16):
                    vf = jnp.where((rb + j >= r0) & (rb + j < r1), 1.0, 0.0)
                    for v in range(NV):
                        out[v] = jnp.maximum(
                            out[v], xb[j, pl.ds(v * 16, 16)] * vf)
                return tuple(out)

            maxv = lax.fori_loop(0, nb, bbody, (zeros,) * NV)
            for v in range(NV):
                ob[pl.ds(v * 16, 16)] = maxv[v]
            pltpu.sync_copy(ob, out_hbm.at[pl.ds(g * W, W)])

    return pl.kernel(
        body,
        out_type=jax.ShapeDtypeStruct((128 * W,), jnp.float32),
        mesh=mesh,
        compiler_params=pltpu.CompilerParams(
            needs_layout_passes=False, use_tc_tiling_on_sc=False),
        scratch_types=[
            pltpu.VMEM((16,), jnp.int32),
            pltpu.VMEM((16, W), jnp.float32),
            pltpu.VMEM((W,), jnp.float32),
            pltpu.SemaphoreType.DMA,
        ],
    )


def _head_kernel(xm_ref, mm_ref, tp_ref, esm_ref, *refs):
    o_ref = refs[-1]
    (mf0w, mf0b, mf1w, mf1b, of0w, of0b, of1w, of1b,
     pf0w, pf0b, pf1w, pf1b, ef0w, ef0b, ef1w, ef1b,
     a0w, a0b, a1w, a1b, c0w, c0b, c1w, c1b, c2w, c2b) = [r[...] for r in refs[:-1]]

    def lin(z, w, b):
        return jnp.dot(z, w, preferred_element_type=jnp.float32) + b

    x = lin(jnp.maximum(xm_ref[...], 0.0), mf0w, mf0b)
    x = lin(jnp.maximum(x, 0.0), mf1w, mf1b)
    m = lin(jnp.maximum(mm_ref[...], 0.0), of0w, of0b)
    m = lin(jnp.maximum(m, 0.0), of1w, of1b)
    t = lin(jnp.maximum(tp_ref[...], 0.0), pf0w, pf0b)
    t = lin(jnp.maximum(t, 0.0), pf1w, pf1b)
    e = lin(jnp.maximum(esm_ref[...], 0.0), ef0w, ef0b)
    e = lin(e, ef1w, ef1b)
    fd = jnp.concatenate([x, m], axis=1)
    fp = jnp.concatenate([t, e], axis=1)

    def att(z):
        return lin(jnp.maximum(lin(z, a0w, a0b), 0.0), a1w, a1b)

    w1 = jax.nn.sigmoid(att(fd + fp))
    f1 = fd * w1 + fp * (1.0 - w1)
    w2 = jax.nn.sigmoid(att(f1))
    f2 = fd * w2 + fp * (1.0 - w2)
    c = jnp.maximum(lin(f2, c0w, c0b), 0.0)
    c = jnp.maximum(lin(c, c1w, c1b), 0.0)
    o_ref[...] = lin(c, c2w, c2b)


def _head(xm, mmo, tp, esm, P):
    ws = []
    for name in ("mol_fc", "motif_fc", "pro_fc", "esm_fc", "att", "cls"):
        for wgt, bia in P[name]:
            ws.append(wgt)
            ws.append(bia.reshape(1, -1))
    c2w, c2b = ws[-2], ws[-1]
    ws[-2] = jnp.pad(c2w, ((0, 0), (0, 127)))
    ws[-1] = jnp.pad(c2b, ((0, 0), (0, 127)))
    out = pl.pallas_call(
        _head_kernel,
        out_shape=jax.ShapeDtypeStruct((128, 128), jnp.float32),
    )(xm, mmo, tp, esm, *ws)
    return out[:, :1]


def _block_diag(a):
    H, O = a.shape
    eye = jnp.eye(H, dtype=a.dtype)
    return (a[:, :, None] * eye[:, None, :]).reshape(H * O, H)


def _shift_kernel(a_ref, b_ref, o_ref):
    ma = jnp.max(a_ref[...], axis=0, keepdims=True)
    mb = jnp.max(b_ref[...], axis=0, keepdims=True)
    o_ref[...] = jnp.concatenate([ma, mb], axis=0)


def _shift(es_p, ed_p):
    esr = es_p.reshape(-1, 128)
    edr = ed_p.reshape(-1, 128)
    out = pl.pallas_call(
        _shift_kernel,
        out_shape=jax.ShapeDtypeStruct((2, 128), jnp.float32),
    )(esr, edr)
    return out.reshape(256)


def _edge_prep(ei, N, ew=None):
    NI = -(-N // _ROWS)
    ar = jnp.arange(N, dtype=jnp.int32)
    src = jnp.concatenate([ei[0].astype(jnp.int32), ar])
    dst = jnp.concatenate([ei[1].astype(jnp.int32), ar])
    ops = [dst, src]
    if ew is not None:
        ops.append(jnp.concatenate([ew, jnp.ones((N,), jnp.float32)]))
    sorted_ops = lax.sort(ops, num_keys=1)
    dst_s, src_s = sorted_ops[0], sorted_ops[1]
    bounds = jnp.arange(NI + 1, dtype=jnp.int32) * _ROWS
    nbnd = jnp.searchsorted(dst_s, bounds).astype(jnp.int32)
    ibnd = jnp.zeros((NI, 16), jnp.int32)
    ibnd = ibnd.at[:, 0].set(nbnd[:-1]).at[:, 1].set(nbnd[1:]).reshape(-1)
    src_p = jnp.pad(src_s, (0, 1216))
    dst_p = jnp.pad(dst_s, (0, 1216))
    ew_s = None
    if ew is not None:
        ew_s = jnp.pad(sorted_ops[2], (0, 1216))
    return src_p, dst_p, ibnd, NI, ew_s


def _gat_layer(x, prep, p, out_relu):
    W, a_s, a_d, b = p
    inF, H, O = W.shape
    N = x.shape[0]
    src_s, dst_s, nbnd, NI, _ = prep
    inFp = _ceil_to(inF, 16)
    Npad = NI * _ROWS
    Wflat = W.reshape(inF, H * O)
    Wes = _mm(Wflat, _block_diag(a_s))
    Wed = _mm(Wflat, _block_diag(a_d))
    es = _mm(x, jnp.pad(Wes, ((0, 0), (0, 16 - H))))
    ed = _mm(x, jnp.pad(Wed, ((0, 0), (0, 16 - H))))
    es_p = jnp.pad(es, ((0, Npad - N), (0, 0)))
    ed_p = jnp.pad(ed, ((0, Npad - N), (0, 0))).reshape(-1)
    shift = _shift(es_p, ed_p)
    x_p = jnp.pad(x, ((0, 0), (0, inFp - inF)))
    agg = _gat_edge_kernel(N, H, inFp, NI)(
        es_p, ed_p, shift, x_p, src_s, dst_s, nbnd)
    agg = agg.reshape(Npad, H * inFp)[:N]
    Wp = jnp.pad(W, ((0, inFp - inF), (0, 0), (0, 0)))
    Wcat = jnp.transpose(Wp, (1, 0, 2)).reshape(H * inFp, O) / H
    return _mm(agg, Wcat, b, act="relu" if out_relu else None)


def _gcn_layer(x, prep, p):
    W, b = p
    inF = x.shape[1]
    N = x.shape[0]
    src_s, dst_s, ibnd, NI, ew_s = prep
    inFp = _ceil_to(inF, 16)
    Npad = NI * _ROWS
    dinv = _gcn_deg_kernel(N, NI)(ew_s, dst_s, ibnd)
    x_p = jnp.pad(x, ((0, 0), (0, inFp - inF)))
    agg = _gcn_agg_kernel(N, inFp, NI)(
        dinv.reshape(Npad, 16), x_p, ew_s, src_s, dst_s, ibnd)
    agg = agg.reshape(Npad, inFp)[:N]
    return _mm(agg, jnp.pad(W, ((0, inFp - inF), (0, 0))), b)


def _gmax(x, batch):
    N, C = x.shape
    W = _ceil_to(C, 16)
    x_p = jnp.pad(x, ((0, 32), (0, W - C)))
    gb = jnp.searchsorted(batch.astype(jnp.int32),
                          jnp.arange(129, dtype=jnp.int32)).astype(jnp.int32)
    gbnd = jnp.zeros((128, 16), jnp.int32)
    gbnd = gbnd.at[:, 0].set(gb[:-1]).at[:, 1].set(gb[1:]).reshape(-1)
    out = _gmax_kernel(N, W)(x_p, gbnd)
    return out.reshape(128, W)[:, :C]


def kernel(mol_x, motif_x, pro_x, pro_edge_weight, pro_emb, edge_attr, params,
           mol_edge_index, motif_edge_index, pro_edge_index, mol_batch,
           motif_batch, pro_batch):
    P = params

    prep = _edge_prep(mol_edge_index, mol_x.shape[0])
    x = _gat_layer(mol_x, prep, P["mol"][0], out_relu=False)
    x = _gat_layer(x, prep, P["mol"][1], out_relu=True)
    x = _gat_layer(x, prep, P["mol"][2], out_relu=True)
    x = _gmax(x, mol_batch)

    prep = _edge_prep(motif_edge_index, motif_x.shape[0])
    m = _gat_layer(motif_x, prep, P["motif"][0], out_relu=False)
    m = _gat_layer(m, prep, P["motif"][1], out_relu=True)
    m = _gat_layer(m, prep, P["motif"][2], out_relu=True)
    m = _gmax(m, motif_batch)

    prep = _edge_prep(pro_edge_index, pro_x.shape[0], ew=pro_edge_weight)
    t = _gcn_layer(pro_x, prep, P["pro_gcn"])
    t = _gat_layer(t, prep, P["pro"][0], out_relu=True)
    t = _gat_layer(t, prep, P["pro"][1], out_relu=True)
    t = _gmax(t, pro_batch)

    return _head(x, m, t, pro_emb, P)
